# baseline (speedup 1.0000x reference)
import functools

import jax
import jax.numpy as jnp
import numpy as np
from jax import lax
from jax.experimental import pallas as pl
from jax.experimental.pallas import tpu as pltpu
from jax.experimental.pallas import tpu_sc as plsc

B, N, C = 16, 256, 32
P = N * (N - 1) // 2
DEG = 4
E = B * N * DEG
VOCAB_PAD = 128
NN = B * N
NC, NS = 2, 16
NW = NC * NS
ECH = E // NW
ROWS_PER_TILE = NN // NS
CHUNK = 2176
NCH = P // CHUNK


def _lrelu(x):
    return jnp.where(x >= 0, x, 0.01 * x)


def _sc_gather_body(tbl_hbm, idx_hbm, out_hbm, idx_v, rows_v, sem):
    cid = lax.axis_index("c")
    sid = lax.axis_index("s")
    wid = sid * NC + cid
    base = wid * ECH
    pltpu.sync_copy(idx_hbm.at[pl.ds(base, ECH)], idx_v)
    pltpu.async_copy(tbl_hbm.at[idx_v], rows_v, sem).wait()
    pltpu.sync_copy(rows_v, out_hbm.at[pl.ds(base, ECH)])


def _sc_gather(tbl, pair_idx):
    mesh = plsc.VectorSubcoreMesh(core_axis_name="c", subcore_axis_name="s")
    fn = functools.partial(
        pl.kernel,
        out_type=jax.ShapeDtypeStruct((E, 16), jnp.float32),
        mesh=mesh,
        scratch_types=[
            pltpu.VMEM((ECH,), jnp.int32),
            pltpu.VMEM((ECH, 16), jnp.float32),
            pltpu.SemaphoreType.DMA,
        ],
        compiler_params=pltpu.CompilerParams(use_tc_tiling_on_sc=False),
    )(_sc_gather_body)
    return fn(tbl, pair_idx)


def _sc_msg_body(x_hbm, e_hbm, src_hbm, dst_hbm, zero_hbm, out_hbm,
                 src_v, dst_v, xj_v, e_v, acc_sh, sem):
    cid = lax.axis_index("c")
    sid = lax.axis_index("s")
    wid = sid * NC + cid
    base = wid * ECH

    @pl.when(sid == 0)
    def _():
        pltpu.sync_copy(zero_hbm, acc_sh)

    plsc.subcore_barrier()

    pltpu.sync_copy(src_hbm.at[pl.ds(base, ECH)], src_v)
    pltpu.sync_copy(dst_hbm.at[pl.ds(base, ECH)], dst_v)
    pltpu.async_copy(x_hbm.at[src_v], xj_v, sem).wait()
    pltpu.sync_copy(e_hbm.at[pl.ds(base, ECH)], e_v)

    def body(r, carry):
        for h in (0, 16):
            v = xj_v[r, pl.ds(h, 16)] + e_v[r, pl.ds(h, 16)]
            xj_v[r, pl.ds(h, 16)] = jnp.maximum(v, 0.0)
        return carry

    lax.fori_loop(0, ECH, body, 0)

    pltpu.sync_copy(xj_v, acc_sh.at[dst_v], add=True)
    plsc.subcore_barrier()
    pltpu.sync_copy(acc_sh.at[pl.ds(sid * ROWS_PER_TILE, ROWS_PER_TILE)],
                    out_hbm.at[cid, pl.ds(sid * ROWS_PER_TILE, ROWS_PER_TILE)])


def _sc_msg(x, e_edge, src, dst, zeros):
    mesh = plsc.VectorSubcoreMesh(core_axis_name="c", subcore_axis_name="s")
    fn = functools.partial(
        pl.kernel,
        out_type=jax.ShapeDtypeStruct((NC, NN, C), jnp.float32),
        mesh=mesh,
        scratch_types=[
            pltpu.VMEM((ECH,), jnp.int32),
            pltpu.VMEM((ECH,), jnp.int32),
            pltpu.VMEM((ECH, C), jnp.float32),
            pltpu.VMEM((ECH, C), jnp.float32),
            pltpu.VMEM_SHARED((NN, C), jnp.float32),
            pltpu.SemaphoreType.DMA,
        ],
        compiler_params=pltpu.CompilerParams(use_tc_tiling_on_sc=False),
    )(_sc_msg_body)
    return fn(x, e_edge, src, dst, zeros)


def _embed_body(tok_ref, emb_ref, out_ref):
    tok = tok_ref[...]
    iota = lax.broadcasted_iota(jnp.int32, (tok.shape[0], VOCAB_PAD), 1)
    onehot = (tok == iota).astype(jnp.float32)
    out_ref[...] = jnp.dot(onehot, emb_ref[...],
                           preferred_element_type=jnp.float32)


def _embed(tokens_col, emb_pad):
    return pl.pallas_call(
        _embed_body,
        grid=(8,),
        in_specs=[
            pl.BlockSpec((512, 1), lambda g: (g, 0)),
            pl.BlockSpec((VOCAB_PAD, C), lambda g: (0, 0)),
        ],
        out_specs=pl.BlockSpec((512, C), lambda g: (g, 0)),
        out_shape=jax.ShapeDtypeStruct((NN, C), jnp.float32),
    )(tokens_col, emb_pad)


def _edgefeat_body(d_ref, ex_ref, ew_ref, eb_ref, out_ref):
    d = d_ref[...]
    ex = ex_ref[...]
    ew = ew_ref[...]
    out_ref[...] = (ex * ew[0:1, :] + d * ew[1:2, :]
                    + jnp.sqrt(jnp.abs(d)) * ew[2:3, :]
                    + (d * d) * ew[3:4, :] + eb_ref[...])


def _edgefeat(d_col, ex_col, ew, eb2):
    return pl.pallas_call(
        _edgefeat_body,
        grid=(8,),
        in_specs=[
            pl.BlockSpec((2048, 1), lambda g: (g, 0)),
            pl.BlockSpec((2048, 1), lambda g: (g, 0)),
            pl.BlockSpec((4, C), lambda g: (0, 0)),
            pl.BlockSpec((1, C), lambda g: (0, 0)),
        ],
        out_specs=pl.BlockSpec((2048, C), lambda g: (g, 0)),
        out_shape=jax.ShapeDtypeStruct((E, C), jnp.float32),
    )(d_col, ex_col, ew, eb2)


def _mlp_body(parts_ref, x_ref, w1n_ref, w1x_ref, b1_ref, w2_ref, b2_ref,
              out_ref):
    n = parts_ref[0] + parts_ref[1]
    h = _lrelu(jnp.dot(n, w1n_ref[...], preferred_element_type=jnp.float32)
               + jnp.dot(x_ref[...], w1x_ref[...],
                         preferred_element_type=jnp.float32)
               + b1_ref[...])
    out_ref[...] = jnp.dot(h, w2_ref[...],
                           preferred_element_type=jnp.float32) + b2_ref[...]


def _mlp3_body(parts_ref, x_ref, w1n_ref, w1x_ref, b1_ref, w2_ref, b2_ref,
               fa_ref, fb_ref, a_ref, bm_ref):
    n = parts_ref[0] + parts_ref[1]
    h = _lrelu(jnp.dot(n, w1n_ref[...], preferred_element_type=jnp.float32)
               + jnp.dot(x_ref[...], w1x_ref[...],
                         preferred_element_type=jnp.float32)
               + b1_ref[...])
    x3 = jnp.dot(h, w2_ref[...],
                 preferred_element_type=jnp.float32) + b2_ref[...]
    a_ref[...] = jnp.dot(x3, fa_ref[...], preferred_element_type=jnp.float32)
    bm_ref[...] = jnp.dot(x3, fb_ref[...], preferred_element_type=jnp.float32)


def _node_mlp(parts, x, w1n, w1x, b12, w2, b22):
    return pl.pallas_call(
        _mlp_body,
        grid=(8,),
        in_specs=[
            pl.BlockSpec((NC, 512, C), lambda g: (0, g, 0)),
            pl.BlockSpec((512, C), lambda g: (g, 0)),
            pl.BlockSpec((C, C), lambda g: (0, 0)),
            pl.BlockSpec((C, C), lambda g: (0, 0)),
            pl.BlockSpec((1, C), lambda g: (0, 0)),
            pl.BlockSpec((C, C), lambda g: (0, 0)),
            pl.BlockSpec((1, C), lambda g: (0, 0)),
        ],
        out_specs=pl.BlockSpec((512, C), lambda g: (g, 0)),
        out_shape=jax.ShapeDtypeStruct((NN, C), jnp.float32),
    )(parts, x, w1n, w1x, b12, w2, b22)


def _node_mlp3(parts, x, w1n, w1x, b12, w2, b22, fw1a, fw1b):
    return pl.pallas_call(
        _mlp3_body,
        grid=(8,),
        in_specs=[
            pl.BlockSpec((NC, 512, C), lambda g: (0, g, 0)),
            pl.BlockSpec((512, C), lambda g: (g, 0)),
            pl.BlockSpec((C, C), lambda g: (0, 0)),
            pl.BlockSpec((C, C), lambda g: (0, 0)),
            pl.BlockSpec((1, C), lambda g: (0, 0)),
            pl.BlockSpec((C, C), lambda g: (0, 0)),
            pl.BlockSpec((1, C), lambda g: (0, 0)),
            pl.BlockSpec((C, C), lambda g: (0, 0)),
            pl.BlockSpec((C, C), lambda g: (0, 0)),
        ],
        out_specs=[
            pl.BlockSpec((512, C), lambda g: (g, 0)),
            pl.BlockSpec((512, C), lambda g: (g, 0)),
        ],
        out_shape=[
            jax.ShapeDtypeStruct((NN, C), jnp.float32),
            jax.ShapeDtypeStruct((NN, C), jnp.float32),
        ],
    )(parts, x, w1n, w1x, b12, w2, b22, fw1a, fw1b)


def _pair_body(a_ref, bm_ref, iu_ref, ju_ref, d_ref, ex_ref, w4_ref, c0_ref,
               fw2_ref, fb2_ref, out_ref):
    iu = iu_ref[...]
    ju = ju_ref[...]
    iota = lax.broadcasted_iota(jnp.int32, (CHUNK, N), 1)
    oh_i = (iu == iota).astype(jnp.float32)
    oh_j = (ju == iota).astype(jnp.float32)
    lhs = jnp.dot(oh_i, a_ref[0], preferred_element_type=jnp.float32)
    rhs = jnp.dot(oh_j, bm_ref[0], preferred_element_type=jnp.float32)
    d = d_ref[...]
    ex = ex_ref[...]
    w4 = w4_ref[...]
    econ = (ex * w4[0:1, :] + d * w4[1:2, :]
            + jnp.sqrt(jnp.abs(d)) * w4[2:3, :] + (d * d) * w4[3:4, :])
    h = _lrelu(lhs + rhs + econ + c0_ref[...])
    s = jnp.sum(h * fw2_ref[...], axis=1, keepdims=True) + fb2_ref[...]
    out_ref[...] = s


def _pair_stage(a3, bm3, iu_col, ju_col, d_col, ex_col, w4, c02, fw2r, fb2r):
    return pl.pallas_call(
        _pair_body,
        grid=(B * NCH,),
        in_specs=[
            pl.BlockSpec((1, N, C), lambda g: (g // NCH, 0, 0)),
            pl.BlockSpec((1, N, C), lambda g: (g // NCH, 0, 0)),
            pl.BlockSpec((CHUNK, 1), lambda g: (g % NCH, 0)),
            pl.BlockSpec((CHUNK, 1), lambda g: (g % NCH, 0)),
            pl.BlockSpec((CHUNK, 1), lambda g: (g, 0)),
            pl.BlockSpec((CHUNK, 1), lambda g: (g, 0)),
            pl.BlockSpec((4, C), lambda g: (0, 0)),
            pl.BlockSpec((1, C), lambda g: (0, 0)),
            pl.BlockSpec((1, C), lambda g: (0, 0)),
            pl.BlockSpec((1, 1), lambda g: (0, 0)),
        ],
        out_specs=pl.BlockSpec((CHUNK, 1), lambda g: (g, 0)),
        out_shape=jax.ShapeDtypeStruct((B * P, 1), jnp.float32),
    )(a3, bm3, iu_col, ju_col, d_col, ex_col, w4, c02, fw2r, fb2r)


def kernel(atom_tokens, d_sims, edge_exists, edge_index, params):
    emb = params["emb"]
    ew, eb = params["edge_lin"]
    (fw1, fb1), (fw2, fb2) = params["final"]

    src = edge_index[:, 0].astype(jnp.int32)
    dst = edge_index[:, 1].astype(jnp.int32)
    a = jnp.minimum(src, dst)
    b = jnp.maximum(src, dst)
    g = a // N
    al = a % N
    bl = b % N
    pair_idx = al * (2 * N - al - 3) // 2 + bl - 1 + g * P

    d_flat = d_sims.reshape(B * P)
    ex_flat = edge_exists.reshape(B * P)
    tbl = jnp.zeros((B * P, 16), jnp.float32)
    tbl = tbl.at[:, 0].set(d_flat).at[:, 1].set(ex_flat)

    de = _sc_gather(tbl, pair_idx)
    d_e = de[:, 0:1]
    ex_e = de[:, 1:2]

    emb_pad = jnp.zeros((VOCAB_PAD, C), jnp.float32).at[:100].set(emb)
    x = _embed(atom_tokens.reshape(NN, 1).astype(jnp.int32), emb_pad)

    e_edge = _edgefeat(d_e, ex_e, ew, eb.reshape(1, C))

    zeros = jnp.zeros((NN, C), jnp.float32)
    fw1a = fw1[:C]
    fw1b = fw1[C:2 * C]
    fw1e = fw1[2 * C:]
    for l, ((w1, b1), (w2, b2)) in enumerate(params["convs"]):
        parts = _sc_msg(x, e_edge, src, dst, zeros)
        w1n, w1x = w1[:C], w1[C:]
        if l < 2:
            x = _node_mlp(parts, x, w1n, w1x, b1.reshape(1, C), w2,
                          b2.reshape(1, C))
        else:
            a3, bm3 = _node_mlp3(parts, x, w1n, w1x, b1.reshape(1, C), w2,
                                 b2.reshape(1, C), fw1a, fw1b)

    w4 = ew @ fw1e
    c02 = (eb @ fw1e + fb1).reshape(1, C)

    iu_np, ju_np = np.triu_indices(N, k=1)
    iu_col = jnp.asarray(iu_np.astype(np.int32).reshape(P, 1))
    ju_col = jnp.asarray(ju_np.astype(np.int32).reshape(P, 1))

    scores = _pair_stage(
        a3.reshape(B, N, C), bm3.reshape(B, N, C), iu_col, ju_col,
        d_sims.reshape(B * P, 1), edge_exists.reshape(B * P, 1),
        w4, c02, fw2.reshape(1, C), fb2.reshape(1, 1))
    return scores.reshape(B, P)


# speedup vs baseline: 11.4805x; 11.4805x over previous
import functools

import jax
import jax.numpy as jnp
import numpy as np
from jax import lax
from jax.experimental import pallas as pl
from jax.experimental.pallas import tpu as pltpu
from jax.experimental.pallas import tpu_sc as plsc

B, N, C = 16, 256, 32
P = N * (N - 1) // 2
DEG = 4
E = B * N * DEG
VOCAB_PAD = 128
NN = B * N
NC, NS = 2, 16
NW = NC * NS
ECH = E // NW
ROWS_PER_TILE = NN // NS
CHUNK = 2176
NCH = P // CHUNK


def _lrelu(x):
    return jnp.where(x >= 0, x, 0.01 * x)


def _sc_gather_body(d_hbm, ex_hbm, idx_hbm, out_hbm, idx_v, dv, exv, sem):
    cid = lax.axis_index("c")
    sid = lax.axis_index("s")
    wid = sid * NC + cid
    base = wid * ECH
    pltpu.sync_copy(idx_hbm.at[pl.ds(base, ECH)], idx_v)
    pltpu.async_copy(d_hbm.at[idx_v], dv, sem).wait()
    pltpu.async_copy(ex_hbm.at[idx_v], exv, sem).wait()
    pltpu.sync_copy(dv, out_hbm.at[0, pl.ds(base, ECH)])
    pltpu.sync_copy(exv, out_hbm.at[1, pl.ds(base, ECH)])


def _sc_gather(d_flat, ex_flat, pair_idx):
    mesh = plsc.VectorSubcoreMesh(core_axis_name="c", subcore_axis_name="s")
    fn = functools.partial(
        pl.kernel,
        out_type=jax.ShapeDtypeStruct((2, E), jnp.float32),
        mesh=mesh,
        scratch_types=[
            pltpu.VMEM((ECH,), jnp.int32),
            pltpu.VMEM((ECH,), jnp.float32),
            pltpu.VMEM((ECH,), jnp.float32),
            pltpu.SemaphoreType.DMA,
        ],
        compiler_params=pltpu.CompilerParams(use_tc_tiling_on_sc=False),
    )(_sc_gather_body)
    return fn(d_flat, ex_flat, pair_idx)


def _sc_msg_body(x_hbm, e_hbm, src_hbm, dst_hbm, zero_hbm, out_hbm,
                 src_v, dst_v, xj_v, e_v, acc_sh, sem):
    cid = lax.axis_index("c")
    sid = lax.axis_index("s")
    wid = sid * NC + cid
    base = wid * ECH

    @pl.when(sid == 0)
    def _():
        pltpu.sync_copy(zero_hbm, acc_sh)

    plsc.subcore_barrier()

    pltpu.sync_copy(src_hbm.at[pl.ds(base, ECH)], src_v)
    pltpu.sync_copy(dst_hbm.at[pl.ds(base, ECH)], dst_v)
    pltpu.async_copy(x_hbm.at[src_v], xj_v, sem).wait()
    pltpu.sync_copy(e_hbm.at[pl.ds(base, ECH)], e_v)

    def body(r, carry):
        for h in (0, 16):
            v = xj_v[r, pl.ds(h, 16)] + e_v[r, pl.ds(h, 16)]
            xj_v[r, pl.ds(h, 16)] = jnp.maximum(v, 0.0)
        return carry

    lax.fori_loop(0, ECH, body, 0)

    pltpu.sync_copy(xj_v, acc_sh.at[dst_v], add=True)
    plsc.subcore_barrier()
    pltpu.sync_copy(acc_sh.at[pl.ds(sid * ROWS_PER_TILE, ROWS_PER_TILE)],
                    out_hbm.at[cid, pl.ds(sid * ROWS_PER_TILE, ROWS_PER_TILE)])


def _sc_msg(x, e_edge, src, dst, zeros):
    mesh = plsc.VectorSubcoreMesh(core_axis_name="c", subcore_axis_name="s")
    fn = functools.partial(
        pl.kernel,
        out_type=jax.ShapeDtypeStruct((NC, NN, C), jnp.float32),
        mesh=mesh,
        scratch_types=[
            pltpu.VMEM((ECH,), jnp.int32),
            pltpu.VMEM((ECH,), jnp.int32),
            pltpu.VMEM((ECH, C), jnp.float32),
            pltpu.VMEM((ECH, C), jnp.float32),
            pltpu.VMEM_SHARED((NN, C), jnp.float32),
            pltpu.SemaphoreType.DMA,
        ],
        compiler_params=pltpu.CompilerParams(use_tc_tiling_on_sc=False),
    )(_sc_msg_body)
    return fn(x, e_edge, src, dst, zeros)


def _embed_body(tok_ref, emb_ref, out_ref):
    tok = tok_ref[...]
    iota = lax.broadcasted_iota(jnp.int32, (tok.shape[0], VOCAB_PAD), 1)
    onehot = (tok == iota).astype(jnp.float32)
    out_ref[...] = jnp.dot(onehot, emb_ref[...],
                           preferred_element_type=jnp.float32)


def _embed(tokens_col, emb_pad):
    return pl.pallas_call(
        _embed_body,
        grid=(8,),
        in_specs=[
            pl.BlockSpec((512, 1), lambda g: (g, 0)),
            pl.BlockSpec((VOCAB_PAD, C), lambda g: (0, 0)),
        ],
        out_specs=pl.BlockSpec((512, C), lambda g: (g, 0)),
        out_shape=jax.ShapeDtypeStruct((NN, C), jnp.float32),
    )(tokens_col, emb_pad)


def _edgefeat_body(d_ref, ex_ref, ew_ref, eb_ref, out_ref):
    d = d_ref[0]
    ex = ex_ref[0]
    feats = jnp.concatenate(
        [ex, d, jnp.sqrt(jnp.abs(d)), d * d], axis=0)
    out_ref[...] = lax.dot_general(
        feats, ew_ref[...], (((0,), (0,)), ((), ())),
        preferred_element_type=jnp.float32) + eb_ref[...]


def _edgefeat(degx, ew, eb2):
    return pl.pallas_call(
        _edgefeat_body,
        grid=(8,),
        in_specs=[
            pl.BlockSpec((1, 1, 2048), lambda g: (g, 0, 0)),
            pl.BlockSpec((1, 1, 2048), lambda g: (8 + g, 0, 0)),
            pl.BlockSpec((4, C), lambda g: (0, 0)),
            pl.BlockSpec((1, C), lambda g: (0, 0)),
        ],
        out_specs=pl.BlockSpec((2048, C), lambda g: (g, 0)),
        out_shape=jax.ShapeDtypeStruct((E, C), jnp.float32),
    )(degx, degx, ew, eb2)


def _mlp_body(parts_ref, x_ref, w1n_ref, w1x_ref, b1_ref, w2_ref, b2_ref,
              out_ref):
    n = parts_ref[0] + parts_ref[1]
    h = _lrelu(jnp.dot(n, w1n_ref[...], preferred_element_type=jnp.float32)
               + jnp.dot(x_ref[...], w1x_ref[...],
                         preferred_element_type=jnp.float32)
               + b1_ref[...])
    out_ref[...] = jnp.dot(h, w2_ref[...],
                           preferred_element_type=jnp.float32) + b2_ref[...]


def _mlp3_body(parts_ref, x_ref, w1n_ref, w1x_ref, b1_ref, w2_ref, b2_ref,
               fa_ref, fb_ref, a_ref, bm_ref):
    n = parts_ref[0] + parts_ref[1]
    h = _lrelu(jnp.dot(n, w1n_ref[...], preferred_element_type=jnp.float32)
               + jnp.dot(x_ref[...], w1x_ref[...],
                         preferred_element_type=jnp.float32)
               + b1_ref[...])
    x3 = jnp.dot(h, w2_ref[...],
                 preferred_element_type=jnp.float32) + b2_ref[...]
    a_ref[...] = jnp.dot(x3, fa_ref[...], preferred_element_type=jnp.float32)
    bm_ref[...] = jnp.dot(x3, fb_ref[...], preferred_element_type=jnp.float32)


def _node_mlp(parts, x, w1n, w1x, b12, w2, b22):
    return pl.pallas_call(
        _mlp_body,
        grid=(8,),
        in_specs=[
            pl.BlockSpec((NC, 512, C), lambda g: (0, g, 0)),
            pl.BlockSpec((512, C), lambda g: (g, 0)),
            pl.BlockSpec((C, C), lambda g: (0, 0)),
            pl.BlockSpec((C, C), lambda g: (0, 0)),
            pl.BlockSpec((1, C), lambda g: (0, 0)),
            pl.BlockSpec((C, C), lambda g: (0, 0)),
            pl.BlockSpec((1, C), lambda g: (0, 0)),
        ],
        out_specs=pl.BlockSpec((512, C), lambda g: (g, 0)),
        out_shape=jax.ShapeDtypeStruct((NN, C), jnp.float32),
    )(parts, x, w1n, w1x, b12, w2, b22)


def _node_mlp3(parts, x, w1n, w1x, b12, w2, b22, fw1a, fw1b):
    return pl.pallas_call(
        _mlp3_body,
        grid=(8,),
        in_specs=[
            pl.BlockSpec((NC, 512, C), lambda g: (0, g, 0)),
            pl.BlockSpec((512, C), lambda g: (g, 0)),
            pl.BlockSpec((C, C), lambda g: (0, 0)),
            pl.BlockSpec((C, C), lambda g: (0, 0)),
            pl.BlockSpec((1, C), lambda g: (0, 0)),
            pl.BlockSpec((C, C), lambda g: (0, 0)),
            pl.BlockSpec((1, C), lambda g: (0, 0)),
            pl.BlockSpec((C, C), lambda g: (0, 0)),
            pl.BlockSpec((C, C), lambda g: (0, 0)),
        ],
        out_specs=[
            pl.BlockSpec((512, C), lambda g: (g, 0)),
            pl.BlockSpec((512, C), lambda g: (g, 0)),
        ],
        out_shape=[
            jax.ShapeDtypeStruct((NN, C), jnp.float32),
            jax.ShapeDtypeStruct((NN, C), jnp.float32),
        ],
    )(parts, x, w1n, w1x, b12, w2, b22, fw1a, fw1b)


BG = 4
NBG = B // BG
MROWS = BG * C
KTOT = 2 * N + 4 * BG


def _pair_body(ab_ref, iu_ref, ju_ref, d_ref, ex_ref, c0_ref, fw2_ref,
               fb2_ref, out_ref):
    iu = iu_ref[0]
    ju = ju_ref[0]
    io = lax.broadcasted_iota(jnp.int32, (2 * N, CHUNK), 0)
    oh = jnp.logical_or(io == iu, (io - N) == ju).astype(jnp.float32)
    d = d_ref[0, 0]
    ex = ex_ref[0, 0]
    rows = []
    for bl in range(BG):
        db = d[bl:bl + 1]
        rows += [ex[bl:bl + 1], db, jnp.sqrt(jnp.abs(db)), db * db]
    ohfeat = jnp.concatenate([oh] + rows, axis=0)
    h = _lrelu(jnp.dot(ab_ref[0], ohfeat,
                       preferred_element_type=jnp.float32) + c0_ref[...])
    hs = (h * fw2_ref[...]).reshape(BG, C, CHUNK)
    out_ref[0, 0] = jnp.sum(hs, axis=1) + fb2_ref[...]


def _pair_stage(ab, iu6, ju6, d6, ex6, c0t, fw2t, fb2r):
    return pl.pallas_call(
        _pair_body,
        grid=(NBG, NCH),
        in_specs=[
            pl.BlockSpec((1, MROWS, KTOT), lambda gb, c: (gb, 0, 0)),
            pl.BlockSpec((1, 1, CHUNK), lambda gb, c: (c, 0, 0)),
            pl.BlockSpec((1, 1, CHUNK), lambda gb, c: (c, 0, 0)),
            pl.BlockSpec((1, 1, BG, CHUNK), lambda gb, c: (gb, c, 0, 0)),
            pl.BlockSpec((1, 1, BG, CHUNK), lambda gb, c: (gb, c, 0, 0)),
            pl.BlockSpec((MROWS, 1), lambda gb, c: (0, 0)),
            pl.BlockSpec((MROWS, 1), lambda gb, c: (0, 0)),
            pl.BlockSpec((1, 1), lambda gb, c: (0, 0)),
        ],
        out_specs=pl.BlockSpec((1, 1, BG, CHUNK), lambda gb, c: (gb, c, 0, 0)),
        out_shape=jax.ShapeDtypeStruct((NBG, NCH, BG, CHUNK), jnp.float32),
    )(ab, iu6, ju6, d6, ex6, c0t, fw2t, fb2r)


def kernel(atom_tokens, d_sims, edge_exists, edge_index, params):
    emb = params["emb"]
    ew, eb = params["edge_lin"]
    (fw1, fb1), (fw2, fb2) = params["final"]

    src = edge_index[:, 0].astype(jnp.int32)
    dst = edge_index[:, 1].astype(jnp.int32)
    a = jnp.minimum(src, dst)
    b = jnp.maximum(src, dst)
    g = a // N
    al = a % N
    bl = b % N
    pair_idx = al * (2 * N - al - 3) // 2 + bl - 1 + g * P

    d_flat = d_sims.reshape(B * P)
    ex_flat = edge_exists.reshape(B * P)
    degx = _sc_gather(d_flat, ex_flat, pair_idx)

    emb_pad = jnp.zeros((VOCAB_PAD, C), jnp.float32).at[:100].set(emb)
    x = _embed(atom_tokens.reshape(NN, 1).astype(jnp.int32), emb_pad)

    e_edge = _edgefeat(degx.reshape(16, 1, 2048), ew, eb.reshape(1, C))

    zeros = jnp.zeros((NN, C), jnp.float32)
    fw1a = fw1[:C]
    fw1b = fw1[C:2 * C]
    fw1e = fw1[2 * C:]
    for l, ((w1, b1), (w2, b2)) in enumerate(params["convs"]):
        parts = _sc_msg(x, e_edge, src, dst, zeros)
        w1n, w1x = w1[:C], w1[C:]
        if l < 2:
            x = _node_mlp(parts, x, w1n, w1x, b1.reshape(1, C), w2,
                          b2.reshape(1, C))
        else:
            a3, bm3 = _node_mlp3(parts, x, w1n, w1x, b1.reshape(1, C), w2,
                                 b2.reshape(1, C), fw1a, fw1b)

    w4 = ew @ fw1e
    c0 = eb @ fw1e + fb1

    iu_np, ju_np = np.triu_indices(N, k=1)
    iu6 = jnp.asarray(iu_np.astype(np.int32).reshape(NCH, 1, CHUNK))
    ju6 = jnp.asarray(ju_np.astype(np.int32).reshape(NCH, 1, CHUNK))

    cat = jnp.concatenate([a3.reshape(B, N, C), bm3.reshape(B, N, C)], axis=1)
    abg = cat.transpose(0, 2, 1).reshape(NBG, MROWS, 2 * N)
    w4s = jnp.kron(jnp.eye(BG, dtype=jnp.float32), w4.T)
    ab = jnp.concatenate(
        [abg, jnp.broadcast_to(w4s[None], (NBG, MROWS, 4 * BG))], axis=2)

    c0t = jnp.tile(c0.reshape(1, C), (BG, 1)).reshape(MROWS, 1)
    fw2t = jnp.tile(fw2.reshape(1, C), (BG, 1)).reshape(MROWS, 1)
    d6 = d_sims.reshape(NBG, BG, NCH, CHUNK).transpose(0, 2, 1, 3)
    ex6 = edge_exists.reshape(NBG, BG, NCH, CHUNK).transpose(0, 2, 1, 3)

    scores = _pair_stage(ab, iu6, ju6, d6, ex6, c0t, fw2t,
                         fb2.reshape(1, 1))
    return scores.transpose(0, 2, 1, 3).reshape(B, P)
